# baseline (speedup 1.0000x reference)
import functools

import jax
import jax.numpy as jnp
from jax import lax
from jax.experimental import pallas as pl
from jax.experimental.pallas import tpu as pltpu
from jax.experimental.pallas import tpu_sc as plsc

_T = 2048
_H = 768
_F = 1536
_E = 16
_B = 128
_NB = 32
_PT = _NB * _B
_NC = 2
_NS = 16
_NW = _NC * _NS
_TW = _T // _NW
_GW = 128


def _router_body(x_ref, wr_ref, br_ref, pos_ref, gate_ref, be_ref):
    x = x_ref[...]
    logits = jnp.dot(x, wr_ref[...], preferred_element_type=jnp.float32)
    logits = logits + br_ref[...]
    m = jnp.max(logits, axis=1, keepdims=True)
    ids = lax.broadcasted_iota(jnp.int32, (_T, _E), 1)
    idx = jnp.min(jnp.where(logits == m, ids, _E), axis=1, keepdims=True)
    s = jax.nn.sigmoid(m)
    gate = s / (s + 1e-10)
    oh = (ids == idx).astype(jnp.float32)
    cnt = jnp.sum(oh, axis=0, keepdims=True)
    ri = lax.broadcasted_iota(jnp.int32, (_T, _T), 0)
    ci = lax.broadcasted_iota(jnp.int32, (_T, _T), 1)
    tri = (ci < ri).astype(jnp.float32)
    ranks_all = jnp.dot(tri, oh, preferred_element_type=jnp.float32)
    rank = jnp.sum(ranks_all * oh, axis=1, keepdims=True)
    nblk = jnp.floor((cnt + (_B - 1)) * (1.0 / _B))
    ei = lax.broadcasted_iota(jnp.int32, (_E, _E), 0)
    ej = lax.broadcasted_iota(jnp.int32, (_E, _E), 1)
    ustrict = (ei < ej).astype(jnp.float32)
    blkstart = jnp.dot(nblk, ustrict, preferred_element_type=jnp.float32)
    total = jnp.sum(nblk, axis=1, keepdims=True)
    bs_tok = jnp.sum(oh * blkstart, axis=1, keepdims=True)
    pos = bs_tok * _B + rank
    pos_ref[...] = pos.astype(jnp.int32)
    gate_ref[...] = jnp.broadcast_to(gate, (_T, _GW))
    b_ids = lax.broadcasted_iota(jnp.int32, (_NB, 1), 0).astype(jnp.float32)
    b_eff = jnp.minimum(b_ids, total - 1.0)
    cmp = (blkstart <= b_eff).astype(jnp.float32)
    be = jnp.sum(cmp, axis=1, keepdims=True) - 1.0
    be_ref[...] = be.astype(jnp.int32)


def _router(x, wr, br2):
    return pl.pallas_call(
        _router_body,
        out_shape=(
            jax.ShapeDtypeStruct((_T, 1), jnp.int32),
            jax.ShapeDtypeStruct((_T, _GW), jnp.float32),
            jax.ShapeDtypeStruct((_NB, 1), jnp.int32),
        ),
    )(x, wr, br2)


@functools.cache
def _sc_mesh():
    return plsc.VectorSubcoreMesh(core_axis_name="c", subcore_axis_name="s")


def _dispatch_body(x_hbm, pos_hbm, gate_hbm, xs_hbm, gs_hbm,
                   idx_v, rows_v, gr_v, sem1, sem2):
    wid = lax.axis_index("s") * _NC + lax.axis_index("c")
    base = wid * _TW
    pltpu.sync_copy(pos_hbm.at[pl.ds(base, _TW)], idx_v)
    pltpu.sync_copy(x_hbm.at[pl.ds(base, _TW)], rows_v)
    pltpu.sync_copy(gate_hbm.at[pl.ds(base, _TW)], gr_v)
    a = pltpu.async_copy(rows_v, xs_hbm.at[idx_v], sem1)
    b = pltpu.async_copy(gr_v, gs_hbm.at[idx_v], sem2)
    a.wait()
    b.wait()


def _dispatch(x, pos, gate2):
    f = pl.kernel(
        _dispatch_body,
        mesh=_sc_mesh(),
        out_type=(
            jax.ShapeDtypeStruct((_PT, _H), jnp.float32),
            jax.ShapeDtypeStruct((_PT, _GW), jnp.float32),
        ),
        scratch_types=[
            pltpu.VMEM((_TW,), jnp.int32),
            pltpu.VMEM((_TW, _H), jnp.float32),
            pltpu.VMEM((_TW, _GW), jnp.float32),
            pltpu.SemaphoreType.DMA,
            pltpu.SemaphoreType.DMA,
        ],
    )
    return f(x, pos, gate2)


def _glu_body(be_ref, xs_ref, gs_ref, wg_ref, wu_ref, wd_ref, out_ref):
    xb = xs_ref[...].astype(jnp.bfloat16)
    g = jnp.dot(xb, wg_ref[0].astype(jnp.bfloat16),
                preferred_element_type=jnp.float32)
    u = jnp.dot(xb, wu_ref[0].astype(jnp.bfloat16),
                preferred_element_type=jnp.float32)
    h = g * jax.nn.sigmoid(g) * u
    y = jnp.dot(h.astype(jnp.bfloat16), wd_ref[0].astype(jnp.bfloat16),
                preferred_element_type=jnp.float32)
    out_ref[...] = y * gs_ref[:, 0:1]


def _glu(be, xs, gs, w_gate, w_up, w_down):
    grid_spec = pltpu.PrefetchScalarGridSpec(
        num_scalar_prefetch=1,
        grid=(_NB,),
        in_specs=[
            pl.BlockSpec((_B, _H), lambda b, be_s: (b, 0)),
            pl.BlockSpec((_B, _GW), lambda b, be_s: (b, 0)),
            pl.BlockSpec((1, _H, _F), lambda b, be_s: (be_s[b], 0, 0)),
            pl.BlockSpec((1, _H, _F), lambda b, be_s: (be_s[b], 0, 0)),
            pl.BlockSpec((1, _F, _H), lambda b, be_s: (be_s[b], 0, 0)),
        ],
        out_specs=pl.BlockSpec((_B, _H), lambda b, be_s: (b, 0)),
    )
    return pl.pallas_call(
        _glu_body,
        grid_spec=grid_spec,
        out_shape=jax.ShapeDtypeStruct((_PT, _H), jnp.float32),
        compiler_params=pltpu.CompilerParams(
            dimension_semantics=("arbitrary",)),
    )(be, xs, gs, w_gate, w_up, w_down)


def _combine_body(ys_hbm, pos_hbm, out_hbm, idx_v, rows_v, sem):
    wid = lax.axis_index("s") * _NC + lax.axis_index("c")
    base = wid * _TW
    pltpu.sync_copy(pos_hbm.at[pl.ds(base, _TW)], idx_v)
    pltpu.async_copy(ys_hbm.at[idx_v], rows_v, sem).wait()
    pltpu.sync_copy(rows_v, out_hbm.at[pl.ds(base, _TW)])


def _combine(ys, pos):
    f = pl.kernel(
        _combine_body,
        mesh=_sc_mesh(),
        out_type=jax.ShapeDtypeStruct((_T, _H), jnp.float32),
        scratch_types=[
            pltpu.VMEM((_TW,), jnp.int32),
            pltpu.VMEM((_TW, _H), jnp.float32),
            pltpu.SemaphoreType.DMA,
        ],
    )
    return f(ys, pos)


def kernel(x, Wr, br, W_gate, W_up, W_down, training=False):
    pos2, gate2, be2 = _router(x, Wr, br.reshape(1, _E))
    pos = pos2.reshape(_T)
    be = be2.reshape(_NB)
    xs, gs = _dispatch(x, pos, gate2)
    ys = _glu(be, xs, gs, W_gate, W_up, W_down)
    return _combine(ys, pos)


# speedup vs baseline: 1.0422x; 1.0422x over previous
import functools

import jax
import jax.numpy as jnp
from jax import lax
from jax.experimental import pallas as pl
from jax.experimental.pallas import tpu as pltpu
from jax.experimental.pallas import tpu_sc as plsc

_T = 2048
_H = 768
_F = 1536
_E = 16
_B = 128
_NB = 32
_PT = _NB * _B
_NC = 2
_NS = 16
_NW = _NC * _NS
_TW = _T // _NW
_GW = 128


def _router_body(x_ref, wr_ref, br_ref, pos_ref, gate_ref, be_ref):
    x = x_ref[...]
    logits = jnp.dot(x, wr_ref[...], preferred_element_type=jnp.float32)
    logits = logits + br_ref[...]
    m = jnp.max(logits, axis=1, keepdims=True)
    ids = lax.broadcasted_iota(jnp.int32, (_T, _E), 1)
    idx = jnp.min(jnp.where(logits == m, ids, _E), axis=1, keepdims=True)
    s = jax.nn.sigmoid(m)
    gate = s / (s + 1e-10)
    oh = (ids == idx).astype(jnp.float32)
    oht = lax.transpose(oh, (1, 0))
    cs = oht
    k = 1
    while k < _T:
        cs = cs + jnp.concatenate(
            [jnp.zeros((_E, k), jnp.float32), cs[:, :-k]], axis=1)
        k *= 2
    rank_t = jnp.sum(oht * cs, axis=0, keepdims=True) - 1.0
    cnt = cs[:, _T - 1:_T]
    nblk = jnp.floor((cnt + (_B - 1)) * (1.0 / _B))
    ei = lax.broadcasted_iota(jnp.int32, (_E, _E), 0)
    ej = lax.broadcasted_iota(jnp.int32, (_E, _E), 1)
    lstrict = (ej < ei).astype(jnp.float32)
    blkstart = jnp.dot(lstrict, nblk, preferred_element_type=jnp.float32)
    total = jnp.sum(nblk, axis=0, keepdims=True)
    bs_tok = jnp.sum(oht * blkstart, axis=0, keepdims=True)
    pos = bs_tok * _B + rank_t
    pos_ref[...] = pos.astype(jnp.int32)
    gate_ref[...] = jnp.broadcast_to(gate, (_T, _GW))
    b_ids = lax.broadcasted_iota(jnp.int32, (1, _NB), 1).astype(jnp.float32)
    b_eff = jnp.minimum(b_ids, total - 1.0)
    cmp = (blkstart <= b_eff).astype(jnp.float32)
    be = jnp.sum(cmp, axis=0, keepdims=True) - 1.0
    be_ref[...] = jnp.concatenate([be, b_eff], axis=0).astype(jnp.int32)


def _router(x, wr, br2):
    return pl.pallas_call(
        _router_body,
        out_shape=(
            jax.ShapeDtypeStruct((1, _T), jnp.int32),
            jax.ShapeDtypeStruct((_T, _GW), jnp.float32),
            jax.ShapeDtypeStruct((2, _NB), jnp.int32),
        ),
    )(x, wr, br2)


@functools.cache
def _sc_mesh():
    return plsc.VectorSubcoreMesh(core_axis_name="c", subcore_axis_name="s")


def _dispatch_body(x_hbm, pos_hbm, gate_hbm, xs_hbm, gs_hbm,
                   idx_v, rows_v, gr_v, sem1, sem2, sem3):
    wid = lax.axis_index("s") * _NC + lax.axis_index("c")
    base = wid * _TW
    l1 = pltpu.async_copy(pos_hbm.at[pl.ds(base, _TW)], idx_v, sem1)
    l2 = pltpu.async_copy(x_hbm.at[pl.ds(base, _TW)], rows_v, sem2)
    l3 = pltpu.async_copy(gate_hbm.at[pl.ds(base, _TW)], gr_v, sem3)
    l1.wait()
    l2.wait()
    a = pltpu.async_copy(rows_v, xs_hbm.at[idx_v], sem1)
    l3.wait()
    b = pltpu.async_copy(gr_v, gs_hbm.at[idx_v], sem2)
    a.wait()
    b.wait()


def _dispatch(x, pos, gate2):
    f = pl.kernel(
        _dispatch_body,
        mesh=_sc_mesh(),
        out_type=(
            jax.ShapeDtypeStruct((_PT, _H), jnp.float32),
            jax.ShapeDtypeStruct((_PT, _GW), jnp.float32),
        ),
        scratch_types=[
            pltpu.VMEM((_TW,), jnp.int32),
            pltpu.VMEM((_TW, _H), jnp.float32),
            pltpu.VMEM((_TW, _GW), jnp.float32),
            pltpu.SemaphoreType.DMA,
            pltpu.SemaphoreType.DMA,
            pltpu.SemaphoreType.DMA,
        ],
    )
    return f(x, pos, gate2)


def _glu_body(be_ref, xs_ref, gs_ref, wg_ref, wu_ref, wd_ref, out_ref):
    xb = xs_ref[...].astype(jnp.bfloat16)
    g = jnp.dot(xb, wg_ref[0].astype(jnp.bfloat16),
                preferred_element_type=jnp.float32)
    u = jnp.dot(xb, wu_ref[0].astype(jnp.bfloat16),
                preferred_element_type=jnp.float32)
    h = g * jax.nn.sigmoid(g) * u
    y = jnp.dot(h.astype(jnp.bfloat16), wd_ref[0].astype(jnp.bfloat16),
                preferred_element_type=jnp.float32)
    out_ref[...] = y * gs_ref[:, 0:1]


def _glu(be, xs, gs, w_gate, w_up, w_down):
    grid_spec = pltpu.PrefetchScalarGridSpec(
        num_scalar_prefetch=1,
        grid=(_NB,),
        in_specs=[
            pl.BlockSpec((_B, _H), lambda b, be_s: (be_s[1, b], 0)),
            pl.BlockSpec((_B, _GW), lambda b, be_s: (be_s[1, b], 0)),
            pl.BlockSpec((1, _H, _F), lambda b, be_s: (be_s[0, b], 0, 0)),
            pl.BlockSpec((1, _H, _F), lambda b, be_s: (be_s[0, b], 0, 0)),
            pl.BlockSpec((1, _F, _H), lambda b, be_s: (be_s[0, b], 0, 0)),
        ],
        out_specs=pl.BlockSpec((_B, _H), lambda b, be_s: (be_s[1, b], 0)),
    )
    return pl.pallas_call(
        _glu_body,
        grid_spec=grid_spec,
        out_shape=jax.ShapeDtypeStruct((_PT, _H), jnp.float32),
        compiler_params=pltpu.CompilerParams(
            dimension_semantics=("arbitrary",)),
    )(be, xs, gs, w_gate, w_up, w_down)


def _combine_body(ys_hbm, pos_hbm, out_hbm, idx_v, rows_v, sem):
    wid = lax.axis_index("s") * _NC + lax.axis_index("c")
    base = wid * _TW
    pltpu.sync_copy(pos_hbm.at[pl.ds(base, _TW)], idx_v)
    pltpu.async_copy(ys_hbm.at[idx_v], rows_v, sem).wait()
    pltpu.sync_copy(rows_v, out_hbm.at[pl.ds(base, _TW)])


def _combine(ys, pos):
    f = pl.kernel(
        _combine_body,
        mesh=_sc_mesh(),
        out_type=jax.ShapeDtypeStruct((_T, _H), jnp.float32),
        scratch_types=[
            pltpu.VMEM((_TW,), jnp.int32),
            pltpu.VMEM((_TW, _H), jnp.float32),
            pltpu.SemaphoreType.DMA,
        ],
    )
    return f(ys, pos)


def kernel(x, Wr, br, W_gate, W_up, W_down, training=False):
    pos2, gate2, be2 = _router(x, Wr, br.reshape(1, _E))
    pos = pos2.reshape(_T)
    be = be2
    xs, gs = _dispatch(x, pos, gate2)
    ys = _glu(be, xs, gs, W_gate, W_up, W_down)
    return _combine(ys, pos)
